# baseline (device time: 24247 ns/iter reference)
import jax
import jax.numpy as jnp
from jax import lax
from jax.experimental import pallas as pl
from jax.experimental.pallas import tpu as pltpu

W = [256, 256, 256, 128, 128]
C = len(W)
OFF = [sum(W[:i]) for i in range(C)]
EW = 128
FW = sum(W) - EW


def kernel(x, dy):
    m, d = x.shape
    _, f = dy.shape
    fh = f // 2
    dh = d // 2
    assert sum(W) == fh

    FWD = [c for c in range(C) if OFF[c] + W[c] <= FW]

    tdims = (((0,), (0,)), ((), ()))

    def body(x_ref, dy_ref, out_ref, p_ref, xr_ref, r_ref, yr_ref,
             pe_ref, xoc_ref, sx, rx, sy, ry, sxe, rxe):
        px = lax.axis_index("x")
        py = lax.axis_index("y")

        barrier = pltpu.get_barrier_semaphore()
        pl.semaphore_signal(barrier, inc=1, device_id=(1 - px, py),
                            device_id_type=pl.DeviceIdType.MESH)
        pl.semaphore_signal(barrier, inc=1, device_id=(px, 1 - py),
                            device_id_type=pl.DeviceIdType.MESH)
        pl.semaphore_wait(barrier, 2)

        def run(col0):
            oc0 = fh - col0
            x_rdmas = []
            y_rdmas = []

            def ystore(c):
                lo, w = OFF[c], W[c]
                yin = pltpu.make_async_remote_copy(
                    src_ref=r_ref.at[:, lo:lo + w],
                    dst_ref=yr_ref.at[:, lo:lo + w],
                    send_sem=sy.at[c],
                    recv_sem=ry.at[c],
                    device_id=(px, 1 - py),
                    device_id_type=pl.DeviceIdType.MESH,
                )
                yin.wait_recv()
                out_ref[:, oc0 + lo:oc0 + lo + w] = yr_ref[:, lo:lo + w]

            def process(c):
                lo, w = OFF[c], W[c]
                x_rdmas[c].wait_recv()
                red = p_ref[pl.ds(px * dh, dh), lo:lo + w] + xr_ref[:, lo:lo + w]
                if c in FWD:
                    r_ref[:, lo:lo + w] = red
                    y_rdma = pltpu.make_async_remote_copy(
                        src_ref=r_ref.at[:, lo:lo + w],
                        dst_ref=yr_ref.at[:, lo:lo + w],
                        send_sem=sy.at[c],
                        recv_sem=ry.at[c],
                        device_id=(px, 1 - py),
                        device_id_type=pl.DeviceIdType.MESH,
                    )
                    y_rdma.start()
                    y_rdmas.append(y_rdma)
                out_ref[:, col0 + lo:col0 + lo + w] = red

            for c in range(C):
                lo, w = OFF[c], W[c]
                p_ref[:, lo:lo + w] = lax.dot_general(
                    x_ref[...], dy_ref[:, col0 + lo:col0 + lo + w], tdims,
                    preferred_element_type=jnp.float32)
                x_rdma = pltpu.make_async_remote_copy(
                    src_ref=p_ref.at[pl.ds((1 - px) * dh, dh), lo:lo + w],
                    dst_ref=xr_ref.at[:, lo:lo + w],
                    send_sem=sx.at[c],
                    recv_sem=rx.at[c],
                    device_id=(1 - px, py),
                    device_id_type=pl.DeviceIdType.MESH,
                )
                x_rdma.start()
                x_rdmas.append(x_rdma)
                if c >= 1:
                    process(c - 1)
                if c >= 3 and (c - 3) in FWD:
                    ystore(c - 3)

            pe_ref[...] = lax.dot_general(
                x_ref[...], dy_ref[:, oc0 + FW:oc0 + FW + EW], tdims,
                preferred_element_type=jnp.float32)
            pex = pltpu.make_async_remote_copy(
                src_ref=pe_ref.at[pl.ds((1 - px) * dh, dh), :],
                dst_ref=xoc_ref,
                send_sem=sxe,
                recv_sem=rxe,
                device_id=(1 - px, py),
                device_id_type=pl.DeviceIdType.MESH,
            )
            pex.start()
            process(C - 1)

            for c in FWD:
                if c > C - 4:
                    ystore(c)

            pex.wait_recv()
            out_ref[:, oc0 + FW:oc0 + FW + EW] = (
                pe_ref[pl.ds(px * dh, dh), :] + xoc_ref[...])

            pex.wait_send()
            for rr in x_rdmas:
                rr.wait_send()
            for rr in y_rdmas:
                rr.wait_send()

        pl.when(py == 0)(lambda: run(0))
        pl.when(py == 1)(lambda: run(fh))

    return pl.pallas_call(
        body,
        out_shape=jax.ShapeDtypeStruct((dh, f), jnp.float32),
        in_specs=[pl.BlockSpec(memory_space=pltpu.VMEM),
                  pl.BlockSpec(memory_space=pltpu.VMEM)],
        out_specs=pl.BlockSpec(memory_space=pltpu.VMEM),
        scratch_shapes=[
            pltpu.VMEM((d, fh), jnp.float32),
            pltpu.VMEM((dh, fh), jnp.float32),
            pltpu.VMEM((dh, fh), jnp.float32),
            pltpu.VMEM((dh, fh), jnp.float32),
            pltpu.VMEM((d, EW), jnp.float32),
            pltpu.VMEM((dh, EW), jnp.float32),
            pltpu.SemaphoreType.DMA((C,)),
            pltpu.SemaphoreType.DMA((C,)),
            pltpu.SemaphoreType.DMA((C,)),
            pltpu.SemaphoreType.DMA((C,)),
            pltpu.SemaphoreType.DMA,
            pltpu.SemaphoreType.DMA,
        ],
        compiler_params=pltpu.CompilerParams(collective_id=0),
    )(x, dy)
